# baseline (speedup 1.0000x reference)
import functools

import jax
import jax.numpy as jnp
from jax import lax
from jax.experimental import pallas as pl
from jax.experimental.pallas import tpu as pltpu

TOKENS = 8192
CHUNK = 1024
NBUF = 4


def _gate_pipelined(x_hbm, w_ref, o_ref, b0, b1, b2, b3, s0, s1, s2, s3):
    nchunks = TOKENS // CHUNK
    bufs = (b0, b1, b2, b3)
    sems = (s0, s1, s2, s3)

    def chunk_copy(i, slot):
        return pltpu.async_copy(
            x_hbm.at[pl.ds(i * CHUNK, CHUNK), :],
            bufs[slot],
            sems[slot],
            priority=slot % 2)

    for s in range(NBUF):
        chunk_copy(s, s)

    for i in range(nchunks):
        slot = i % NBUF
        pltpu.make_async_copy(
            x_hbm.at[pl.ds(i * CHUNK, CHUNK), :],
            bufs[slot], sems[slot]).wait()
        o_ref[pl.ds(i * CHUNK, CHUNK), :] = lax.dot_general(
            bufs[slot][...], w_ref[...],
            dimension_numbers=(((1,), (1,)), ((), ())),
            preferred_element_type=jnp.float32)
        if i + NBUF < nchunks:
            chunk_copy(i + NBUF, slot)


@jax.jit
def kernel(x, W):
    tokens, model_dim = x.shape
    num_experts = W.shape[0]
    return pl.pallas_call(
        _gate_pipelined,
        in_specs=[
            pl.BlockSpec(memory_space=pltpu.MemorySpace.HBM),
            pl.BlockSpec((num_experts, model_dim), lambda: (0, 0)),
        ],
        out_specs=pl.BlockSpec((tokens, num_experts), lambda: (0, 0)),
        out_shape=jax.ShapeDtypeStruct((tokens, num_experts), jnp.float32),
        scratch_shapes=(
            [pltpu.VMEM((CHUNK, 2048), jnp.float32) for _ in range(NBUF)]
            + [pltpu.SemaphoreType.DMA for _ in range(NBUF)]
        ),
    )(x, W)


# speedup vs baseline: 1.2136x; 1.2136x over previous
import functools

import jax
import jax.numpy as jnp
from jax import lax
from jax.experimental import pallas as pl
from jax.experimental.pallas import tpu as pltpu

TOKEN_BLOCK = 1024


def _gate_block(x_ref, w_ref, o_ref):
    o_ref[...] = lax.dot_general(
        x_ref[...], w_ref[...],
        dimension_numbers=(((1,), (1,)), ((), ())),
        preferred_element_type=jnp.float32)


@jax.jit
def kernel(x, W):
    tokens, model_dim = x.shape
    num_experts = W.shape[0]
    grid = (tokens // TOKEN_BLOCK,)
    return pl.pallas_call(
        _gate_block,
        grid=grid,
        in_specs=[
            pl.BlockSpec((TOKEN_BLOCK, model_dim), lambda i: (i, 0)),
            pl.BlockSpec((num_experts, model_dim), lambda i: (0, 0)),
        ],
        out_specs=pl.BlockSpec((TOKEN_BLOCK, num_experts), lambda i: (i, 0)),
        out_shape=jax.ShapeDtypeStruct((tokens, num_experts), jnp.float32),
        compiler_params=pltpu.CompilerParams(
            dimension_semantics=("parallel",),
            skip_device_barrier=True,
        ),
    )(x, W)


# speedup vs baseline: 1.2152x; 1.0013x over previous
import functools

import jax
import jax.numpy as jnp
from jax import lax
from jax.experimental import pallas as pl
from jax.experimental.pallas import tpu as pltpu

TOKEN_BLOCK = 1024


def _gate_block(x_ref, w_ref, o_ref):
    i = pl.program_id(0)
    o_ref[pl.ds(i * TOKEN_BLOCK, TOKEN_BLOCK), :] = lax.dot_general(
        x_ref[...], w_ref[...],
        dimension_numbers=(((1,), (1,)), ((), ())),
        preferred_element_type=jnp.float32)


@jax.jit
def kernel(x, W):
    tokens, model_dim = x.shape
    num_experts = W.shape[0]
    grid = (tokens // TOKEN_BLOCK,)
    return pl.pallas_call(
        _gate_block,
        grid=grid,
        in_specs=[
            pl.BlockSpec((TOKEN_BLOCK, model_dim), lambda i: (i, 0)),
            pl.BlockSpec((num_experts, model_dim), lambda i: (0, 0)),
        ],
        out_specs=pl.BlockSpec((tokens, num_experts), lambda i: (0, 0)),
        out_shape=jax.ShapeDtypeStruct((tokens, num_experts), jnp.float32),
        compiler_params=pltpu.CompilerParams(
            dimension_semantics=("parallel",),
            skip_device_barrier=True,
        ),
    )(x, W)
